# Initial kernel scaffold; baseline (speedup 1.0000x reference)
import jax
import jax.numpy as jnp
from jax.experimental import pallas as pl


def kernel(user_embeddings, item_embeddings, adj_indices, adj_values):
    raise NotImplementedError("write your pallas kernel here")


# speedup vs baseline: 4.7075x; 4.7075x over previous
import functools

import jax
import jax.numpy as jnp
from jax import lax
from jax.experimental import pallas as pl
from jax.experimental.pallas import tpu as pltpu
from jax.experimental.pallas import tpu_sc as plsc

N_USERS = 4000
N_ITEMS = 6000
N = N_USERS + N_ITEMS
N_PAD = 10240
D = 128
E = 320000
N_LAYERS = 3

NC = 2
NS = 16
NW = NC * NS
EPW = E // NW
CHUNK = 80
NCHUNK = EPW // CHUNK
ROWS_PER_TILE = N_PAD // NS
RCHUNK = 80
NRCOPY = ROWS_PER_TILE // RCHUNK


def _sc_layer_body(x_hbm, rows_hbm, cols_hbm, vals_hbm, y_hbm,
                   acc, rows_v, cols_v, vals_v, gbuf, sem):
    cid = lax.axis_index("c")
    sid = lax.axis_index("s")
    wid = cid * NS + sid

    pltpu.sync_copy(rows_hbm.at[wid], rows_v)
    pltpu.sync_copy(cols_hbm.at[wid], cols_v)
    pltpu.sync_copy(vals_hbm.at[pl.ds(wid * EPW, EPW)], vals_v)

    zero16 = jnp.zeros((16,), jnp.float32)

    def _zero_row(i, _):
        for c in range(D // 16):
            gbuf[i, pl.ds(c * 16, 16)] = zero16
        return 0

    lax.fori_loop(0, RCHUNK, _zero_row, 0)
    for t in range(NRCOPY):
        pltpu.sync_copy(gbuf, acc.at[pl.ds(sid * ROWS_PER_TILE + t * RCHUNK,
                                           RCHUNK)])
    plsc.subcore_barrier()

    def _chunk(j, _):
        pltpu.async_copy(x_hbm.at[cols_v.at[j]], gbuf, sem).wait()

        def _scale_row(k, _):
            val = plsc.load_gather(
                vals_v, [jnp.full((16,), j * CHUNK + k, jnp.int32)])
            for c in range(D // 16):
                sl = pl.ds(c * 16, 16)
                gbuf[k, sl] = gbuf[k, sl] * val
            return 0

        lax.fori_loop(0, CHUNK, _scale_row, 0)
        pltpu.sync_copy(gbuf, acc.at[rows_v.at[j]], add=True)
        return 0

    lax.fori_loop(0, NCHUNK, _chunk, 0)
    plsc.subcore_barrier()

    for t in range(NRCOPY):
        start = sid * ROWS_PER_TILE + t * RCHUNK
        pltpu.sync_copy(acc.at[pl.ds(start, RCHUNK)], gbuf)
        pltpu.sync_copy(gbuf, y_hbm.at[cid, pl.ds(start, RCHUNK)])


def _sc_layer(x, rows3, cols3, vals1):
    mesh = plsc.VectorSubcoreMesh(core_axis_name="c", subcore_axis_name="s",
                                  num_cores=NC, num_subcores=NS)
    return pl.kernel(
        _sc_layer_body,
        out_type=jax.ShapeDtypeStruct((NC, N_PAD, D), jnp.float32),
        mesh=mesh,
        scratch_types=[
            pltpu.VMEM_SHARED((N_PAD, D), jnp.float32),
            pltpu.VMEM((NCHUNK, CHUNK), jnp.int32),
            pltpu.VMEM((NCHUNK, CHUNK), jnp.int32),
            pltpu.VMEM((EPW,), jnp.float32),
            pltpu.VMEM((CHUNK, D), jnp.float32),
            pltpu.SemaphoreType.DMA,
        ],
        compiler_params=pltpu.CompilerParams(needs_layout_passes=False,
                                             use_tc_tiling_on_sc=False),
        name="lightgcn_sc_layer",
    )(x, rows3, cols3, vals1)


def _combine_body(scale, p_ref, s_ref, x_ref, sout_ref):
    x = p_ref[0] + p_ref[1]
    x_ref[...] = x
    sout_ref[...] = (s_ref[...] + x) * scale


def _combine(p, s, scale):
    blk = 1024
    grid = (N_PAD // blk,)
    return pl.pallas_call(
        functools.partial(_combine_body, scale),
        grid=grid,
        in_specs=[
            pl.BlockSpec((NC, blk, D), lambda i: (0, i, 0)),
            pl.BlockSpec((blk, D), lambda i: (i, 0)),
        ],
        out_specs=[
            pl.BlockSpec((blk, D), lambda i: (i, 0)),
            pl.BlockSpec((blk, D), lambda i: (i, 0)),
        ],
        out_shape=[
            jax.ShapeDtypeStruct((N_PAD, D), jnp.float32),
            jax.ShapeDtypeStruct((N_PAD, D), jnp.float32),
        ],
        name="lightgcn_combine",
    )(p, s)


def kernel(user_embeddings, item_embeddings, adj_indices, adj_values):
    x0 = jnp.concatenate(
        [user_embeddings, item_embeddings,
         jnp.zeros((N_PAD - N, D), jnp.float32)], axis=0)
    rows3 = adj_indices[0].astype(jnp.int32).reshape(NW, NCHUNK, CHUNK)
    cols3 = adj_indices[1].astype(jnp.int32).reshape(NW, NCHUNK, CHUNK)
    vals1 = adj_values

    x = x0
    s = x0
    for layer in range(N_LAYERS):
        p = _sc_layer(x, rows3, cols3, vals1)
        scale = 1.0 / (N_LAYERS + 1) if layer == N_LAYERS - 1 else 1.0
        x, s = _combine(p, s, scale)
    return s[:N_USERS], s[N_USERS:N]


# speedup vs baseline: 5.1451x; 1.0930x over previous
import functools

import jax
import jax.numpy as jnp
from jax import lax
from jax.experimental import pallas as pl
from jax.experimental.pallas import tpu as pltpu
from jax.experimental.pallas import tpu_sc as plsc

N_USERS = 4000
N_ITEMS = 6000
N = N_USERS + N_ITEMS
N_PAD = 10240
D = 128
E = 320000
N_LAYERS = 3

NC = 2
NS = 16
NW = NC * NS
EPW = E // NW
CHUNK = 125
NCHUNK = EPW // CHUNK
ROWS_PER_TILE = N_PAD // NS
RCHUNK = 80
NRCOPY = ROWS_PER_TILE // RCHUNK


def _sc_layer_body(x_hbm, rows_hbm, cols_hbm, vals_hbm, y_hbm,
                   acc, rows_v, cols_v, vals_v, gbuf, sem):
    cid = lax.axis_index("c")
    sid = lax.axis_index("s")
    wid = cid * NS + sid

    pltpu.sync_copy(rows_hbm.at[wid], rows_v)
    pltpu.sync_copy(cols_hbm.at[wid], cols_v)
    pltpu.sync_copy(vals_hbm.at[pl.ds(wid * EPW, EPW)], vals_v)

    zero16 = jnp.zeros((16,), jnp.float32)

    def _zero_row(i, _):
        for c in range(D // 16):
            gbuf[i, pl.ds(c * 16, 16)] = zero16
        return 0

    lax.fori_loop(0, RCHUNK, _zero_row, 0)
    for t in range(NRCOPY):
        pltpu.sync_copy(gbuf.at[pl.ds(0, RCHUNK)],
                        acc.at[pl.ds(sid * ROWS_PER_TILE + t * RCHUNK,
                                     RCHUNK)])
    plsc.subcore_barrier()

    def _chunk(j, _):
        pltpu.async_copy(x_hbm.at[cols_v.at[j]], gbuf, sem).wait()

        def _scale_row(k, _):
            val = plsc.load_gather(
                vals_v, [jnp.full((16,), j * CHUNK + k, jnp.int32)])
            for c in range(D // 16):
                sl = pl.ds(c * 16, 16)
                gbuf[k, sl] = gbuf[k, sl] * val
            return 0

        lax.fori_loop(0, CHUNK, _scale_row, 0)
        pltpu.sync_copy(gbuf, acc.at[rows_v.at[j]], add=True)
        return 0

    lax.fori_loop(0, NCHUNK, _chunk, 0)
    plsc.subcore_barrier()

    for t in range(NRCOPY):
        start = sid * ROWS_PER_TILE + t * RCHUNK
        pltpu.sync_copy(acc.at[pl.ds(start, RCHUNK)], gbuf.at[pl.ds(0, RCHUNK)])
        pltpu.sync_copy(gbuf.at[pl.ds(0, RCHUNK)],
                        y_hbm.at[cid, pl.ds(start, RCHUNK)])


def _sc_layer(x, rows3, cols3, vals1):
    mesh = plsc.VectorSubcoreMesh(core_axis_name="c", subcore_axis_name="s",
                                  num_cores=NC, num_subcores=NS)
    return pl.kernel(
        _sc_layer_body,
        out_type=jax.ShapeDtypeStruct((NC, N_PAD, D), jnp.float32),
        mesh=mesh,
        scratch_types=[
            pltpu.VMEM_SHARED((N_PAD, D), jnp.float32),
            pltpu.VMEM((NCHUNK, CHUNK), jnp.int32),
            pltpu.VMEM((NCHUNK, CHUNK), jnp.int32),
            pltpu.VMEM((EPW,), jnp.float32),
            pltpu.VMEM((CHUNK, D), jnp.float32),
            pltpu.SemaphoreType.DMA,
        ],
        compiler_params=pltpu.CompilerParams(needs_layout_passes=False,
                                             use_tc_tiling_on_sc=False),
        name="lightgcn_sc_layer",
    )(x, rows3, cols3, vals1)


def _combine_body(scale, p_ref, s_ref, x_ref, sout_ref):
    x = p_ref[0] + p_ref[1]
    x_ref[...] = x
    sout_ref[...] = (s_ref[...] + x) * scale


def _combine(p, s, scale):
    blk = 1024
    grid = (N_PAD // blk,)
    return pl.pallas_call(
        functools.partial(_combine_body, scale),
        grid=grid,
        in_specs=[
            pl.BlockSpec((NC, blk, D), lambda i: (0, i, 0)),
            pl.BlockSpec((blk, D), lambda i: (i, 0)),
        ],
        out_specs=[
            pl.BlockSpec((blk, D), lambda i: (i, 0)),
            pl.BlockSpec((blk, D), lambda i: (i, 0)),
        ],
        out_shape=[
            jax.ShapeDtypeStruct((N_PAD, D), jnp.float32),
            jax.ShapeDtypeStruct((N_PAD, D), jnp.float32),
        ],
        name="lightgcn_combine",
    )(p, s)


def kernel(user_embeddings, item_embeddings, adj_indices, adj_values):
    x0 = jnp.concatenate(
        [user_embeddings, item_embeddings,
         jnp.zeros((N_PAD - N, D), jnp.float32)], axis=0)
    rows3 = adj_indices[0].astype(jnp.int32).reshape(NW, NCHUNK, CHUNK)
    cols3 = adj_indices[1].astype(jnp.int32).reshape(NW, NCHUNK, CHUNK)
    vals1 = adj_values

    x = x0
    s = x0
    for layer in range(N_LAYERS):
        p = _sc_layer(x, rows3, cols3, vals1)
        scale = 1.0 / (N_LAYERS + 1) if layer == N_LAYERS - 1 else 1.0
        x, s = _combine(p, s, scale)
    return s[:N_USERS], s[N_USERS:N]


# speedup vs baseline: 7.8788x; 1.5313x over previous
import functools

import jax
import jax.numpy as jnp
from jax import lax
from jax.experimental import pallas as pl
from jax.experimental.pallas import tpu as pltpu
from jax.experimental.pallas import tpu_sc as plsc

N_USERS = 4000
N_ITEMS = 6000
N = N_USERS + N_ITEMS
D = 128
E = 320000
N_LAYERS = 3

NC = 2
NS = 16
NW = NC * NS
EPW = E // NW
CHUNK = 80
NCHUNK = EPW // CHUNK
ROWS_PER_TILE = N // NS
RCHUNK = 80
NRCOPY = ROWS_PER_TILE // RCHUNK
RTAIL = ROWS_PER_TILE - NRCOPY * RCHUNK


def _sc_layer_body(x_hbm, rows_hbm, cols_hbm, vals_hbm, y_hbm,
                   acc, rows_v, cols_v, vals_v, gbuf_a, gbuf_b,
                   sem_a, sem_b):
    cid = lax.axis_index("c")
    sid = lax.axis_index("s")
    wid = cid * NS + sid

    pltpu.sync_copy(rows_hbm.at[wid], rows_v)
    pltpu.sync_copy(cols_hbm.at[wid], cols_v)
    pltpu.sync_copy(vals_hbm.at[pl.ds(wid * EPW, EPW)], vals_v)

    zero16 = jnp.zeros((16,), jnp.float32)

    def _zero_row(i, _):
        for c in range(D // 16):
            gbuf_a[i, pl.ds(c * 16, 16)] = zero16
        return 0

    lax.fori_loop(0, CHUNK, _zero_row, 0)
    base = sid * ROWS_PER_TILE
    for t in range(NRCOPY):
        pltpu.sync_copy(gbuf_a.at[pl.ds(0, RCHUNK)],
                        acc.at[pl.ds(base + t * RCHUNK, RCHUNK)])
    pltpu.sync_copy(gbuf_a.at[pl.ds(0, RTAIL)],
                    acc.at[pl.ds(base + NRCOPY * RCHUNK, RTAIL)])
    plsc.subcore_barrier()

    def _gather(j, buf, sem):
        return pltpu.make_async_copy(x_hbm.at[cols_v.at[j]], buf, sem)

    def _scale(j, buf):
        def _scale_row(k, _):
            val = plsc.load_gather(
                vals_v, [jnp.full((16,), j * CHUNK + k, jnp.int32)])
            for c in range(D // 16):
                sl = pl.ds(c * 16, 16)
                buf[k, sl] = buf[k, sl] * val
            return 0
        lax.fori_loop(0, CHUNK, _scale_row, 0)

    _gather(0, gbuf_a, sem_a).start()

    def _pair(i, _):
        ja = 2 * i
        jb = 2 * i + 1
        _gather(jb, gbuf_b, sem_b).start()
        _gather(ja, gbuf_a, sem_a).wait()
        _scale(ja, gbuf_a)
        pltpu.sync_copy(gbuf_a, acc.at[rows_v.at[ja]], add=True)
        _gather(ja + 2, gbuf_a, sem_a).start()
        _gather(jb, gbuf_b, sem_b).wait()
        _scale(jb, gbuf_b)
        pltpu.sync_copy(gbuf_b, acc.at[rows_v.at[jb]], add=True)
        return 0

    lax.fori_loop(0, (NCHUNK - 1) // 2, _pair, 0)
    jl = NCHUNK - 1
    _gather(jl, gbuf_a, sem_a).wait()
    _scale(jl, gbuf_a)
    pltpu.sync_copy(gbuf_a, acc.at[rows_v.at[jl]], add=True)
    plsc.subcore_barrier()

    for t in range(NRCOPY):
        start = base + t * RCHUNK
        pltpu.sync_copy(acc.at[pl.ds(start, RCHUNK)],
                        gbuf_a.at[pl.ds(0, RCHUNK)])
        pltpu.sync_copy(gbuf_a.at[pl.ds(0, RCHUNK)],
                        y_hbm.at[cid, pl.ds(start, RCHUNK)])
    start = base + NRCOPY * RCHUNK
    pltpu.sync_copy(acc.at[pl.ds(start, RTAIL)], gbuf_a.at[pl.ds(0, RTAIL)])
    pltpu.sync_copy(gbuf_a.at[pl.ds(0, RTAIL)],
                    y_hbm.at[cid, pl.ds(start, RTAIL)])


def _sc_layer(x, rows3, cols3, vals1):
    mesh = plsc.VectorSubcoreMesh(core_axis_name="c", subcore_axis_name="s",
                                  num_cores=NC, num_subcores=NS)
    return pl.kernel(
        _sc_layer_body,
        out_type=jax.ShapeDtypeStruct((NC, N, D), jnp.float32),
        mesh=mesh,
        scratch_types=[
            pltpu.VMEM_SHARED((N, D), jnp.float32),
            pltpu.VMEM((NCHUNK, CHUNK), jnp.int32),
            pltpu.VMEM((NCHUNK, CHUNK), jnp.int32),
            pltpu.VMEM((EPW,), jnp.float32),
            pltpu.VMEM((CHUNK, D), jnp.float32),
            pltpu.VMEM((CHUNK, D), jnp.float32),
            pltpu.SemaphoreType.DMA,
            pltpu.SemaphoreType.DMA,
        ],
        compiler_params=pltpu.CompilerParams(needs_layout_passes=False,
                                             use_tc_tiling_on_sc=False),
        name="lightgcn_sc_layer",
    )(x, rows3, cols3, vals1)


def _combine_body(scale, p_ref, s_ref, x_ref, sout_ref):
    x = p_ref[0] + p_ref[1]
    x_ref[...] = x
    sout_ref[...] = (s_ref[...] + x) * scale


def _combine(p, s, scale):
    blk = 1000
    grid = (N // blk,)
    return pl.pallas_call(
        functools.partial(_combine_body, scale),
        grid=grid,
        in_specs=[
            pl.BlockSpec((NC, blk, D), lambda i: (0, i, 0)),
            pl.BlockSpec((blk, D), lambda i: (i, 0)),
        ],
        out_specs=[
            pl.BlockSpec((blk, D), lambda i: (i, 0)),
            pl.BlockSpec((blk, D), lambda i: (i, 0)),
        ],
        out_shape=[
            jax.ShapeDtypeStruct((N, D), jnp.float32),
            jax.ShapeDtypeStruct((N, D), jnp.float32),
        ],
        name="lightgcn_combine",
    )(p, s)


def kernel(user_embeddings, item_embeddings, adj_indices, adj_values):
    x0 = jnp.concatenate([user_embeddings, item_embeddings], axis=0)
    rows3 = adj_indices[0].astype(jnp.int32).reshape(NW, NCHUNK, CHUNK)
    cols3 = adj_indices[1].astype(jnp.int32).reshape(NW, NCHUNK, CHUNK)
    vals1 = adj_values

    x = x0
    s = x0
    for layer in range(N_LAYERS):
        p = _sc_layer(x, rows3, cols3, vals1)
        scale = 1.0 / (N_LAYERS + 1) if layer == N_LAYERS - 1 else 1.0
        x, s = _combine(p, s, scale)
    return s[:N_USERS], s[N_USERS:]
